# baseline (speedup 1.0000x reference)
import functools

import jax
import jax.numpy as jnp
from jax import lax
from jax.experimental import pallas as pl
from jax.experimental.pallas import tpu as pltpu
from jax.experimental.pallas import tpu_sc as plsc

N_FIELDS = 26
VOCAB = 100000
EMBED_DIM = 32
BATCH = 16384

NC = 2
NS = 16
L = 16
NW = NC * NS

FP = 32
ROWS = BATCH * FP
RPW = ROWS // NW
GCHUNK = 128
OCHUNK = 1024
NGO = OCHUNK // GCHUNK
NOUTER = RPW // OCHUNK

VB = 16384
NVB = (VOCAB + VB - 1) // VB
FQ = 4
NQ = (N_FIELDS + FQ - 1) // FQ
QW = FQ * EMBED_DIM


def _tc_detile(tab_t2):
    def body(i_ref, o_ref):
        o_ref[0] = i_ref[...].T

    return pl.pallas_call(
        body,
        grid=(NQ, NVB),
        in_specs=[
            pl.BlockSpec((QW, VB), lambda q, k: (q, k)),
        ],
        out_specs=pl.BlockSpec((1, VB, QW), lambda q, k: (q, k, 0)),
        out_shape=jax.ShapeDtypeStruct((NQ, VOCAB, QW), jnp.float32),
    )(tab_t2)


def _sc_gather(idx_flat, table2d):
    mesh = plsc.VectorSubcoreMesh(core_axis_name="c", subcore_axis_name="s")

    @functools.partial(
        pl.kernel,
        out_type=jax.ShapeDtypeStruct((ROWS, EMBED_DIM), jnp.float32),
        mesh=mesh,
        scratch_types=[
            pltpu.VMEM((RPW,), jnp.int32),
            pltpu.VMEM((OCHUNK, EMBED_DIM), jnp.float32),
            pltpu.SemaphoreType.DMA,
        ],
        compiler_params=pltpu.CompilerParams(use_tc_tiling_on_sc=False),
    )
    def k(idx_hbm, table_hbm, out_hbm, idx_v, rows_v, sem):
        wid = lax.axis_index("s") * NC + lax.axis_index("c")
        base = wid * RPW

        pltpu.sync_copy(idx_hbm.at[pl.ds(base, RPW)], idx_v)

        lane = lax.iota(jnp.int32, L)

        def fix(i, carry):
            p = i * L + lane
            f = lax.min(lax.rem(p, FP), N_FIELDS - 1)
            sl = pl.ds(i * L, L)
            idx_v[sl] = (
                idx_v[sl] * FQ
                + lax.div(f, FQ) * (FQ * VOCAB)
                + lax.rem(f, FQ)
            )
            return carry

        lax.fori_loop(0, RPW // L, fix, 0)

        def outer(c, carry):
            row0 = c * OCHUNK
            copies = []
            for g in range(NGO):
                src = table_hbm.at[idx_v.at[pl.ds(row0 + g * GCHUNK, GCHUNK)]]
                dst = rows_v.at[pl.ds(g * GCHUNK, GCHUNK), :]
                copies.append(pltpu.async_copy(src, dst, sem))
            for cp in copies:
                cp.wait()
            pltpu.sync_copy(rows_v, out_hbm.at[pl.ds(base + row0, OCHUNK), :])
            return carry

        lax.fori_loop(0, NOUTER, outer, 0)

    return k(idx_flat, table2d)


def kernel(sparse_inputs, tables):
    idx = jnp.pad(
        sparse_inputs.astype(jnp.int32), ((0, 0), (0, FP - N_FIELDS))
    ).reshape(ROWS)
    tab_t2 = jnp.transpose(tables, (0, 2, 1)).reshape(
        N_FIELDS * EMBED_DIM, VOCAB
    )
    tab_lin = _tc_detile(tab_t2).reshape(NQ * VOCAB * FQ, EMBED_DIM)
    out = _sc_gather(idx, tab_lin)
    return out.reshape(BATCH, FP, EMBED_DIM)[:, :N_FIELDS, :]


# speedup vs baseline: 2.8895x; 2.8895x over previous
import functools

import jax
import jax.numpy as jnp
from jax import lax
from jax.experimental import pallas as pl
from jax.experimental.pallas import tpu as pltpu
from jax.experimental.pallas import tpu_sc as plsc

N_FIELDS = 26
VOCAB = 100000
EMBED_DIM = 32
BATCH = 16384

NC = 2
NS = 16
L = 16
NW = NC * NS

ROWS = BATCH * N_FIELDS
RPW = ROWS // NW
GCHUNK = 128
OCHUNK = 1024
NGO = OCHUNK // GCHUNK
NOUTER = RPW // OCHUNK

VB = 16384
NVB = (VOCAB + VB - 1) // VB
FQ = 4
NQ = (N_FIELDS + FQ - 1) // FQ
QW = FQ * EMBED_DIM


def _tc_detile(tab_t2):
    def body(i_ref, o_ref):
        o_ref[0] = i_ref[...].T

    return pl.pallas_call(
        body,
        grid=(NQ, NVB),
        in_specs=[
            pl.BlockSpec((QW, VB), lambda q, k: (q, k)),
        ],
        out_specs=pl.BlockSpec((1, VB, QW), lambda q, k: (q, k, 0)),
        out_shape=jax.ShapeDtypeStruct((NQ, VOCAB, QW), jnp.float32),
    )(tab_t2)


def _sc_gather(idx_flat, table2d):
    mesh = plsc.VectorSubcoreMesh(core_axis_name="c", subcore_axis_name="s")

    @functools.partial(
        pl.kernel,
        out_type=jax.ShapeDtypeStruct((ROWS, EMBED_DIM), jnp.float32),
        mesh=mesh,
        scratch_types=[
            pltpu.VMEM((RPW,), jnp.int32),
            pltpu.VMEM((OCHUNK, EMBED_DIM), jnp.float32),
            pltpu.SemaphoreType.DMA,
        ],
        compiler_params=pltpu.CompilerParams(use_tc_tiling_on_sc=False),
    )
    def k(idx_hbm, table_hbm, out_hbm, idx_v, rows_v, sem):
        wid = lax.axis_index("s") * NC + lax.axis_index("c")
        base = wid * RPW

        pltpu.sync_copy(idx_hbm.at[pl.ds(base, RPW)], idx_v)

        lane = lax.iota(jnp.int32, L)

        def fix(i, carry):
            p = i * L + lane
            f = lax.rem(p, N_FIELDS)
            sl = pl.ds(i * L, L)
            idx_v[sl] = (
                idx_v[sl] * FQ
                + lax.div(f, FQ) * (FQ * VOCAB)
                + lax.rem(f, FQ)
            )
            return carry

        lax.fori_loop(0, RPW // L, fix, 0)

        def outer(c, carry):
            row0 = c * OCHUNK
            copies = []
            for g in range(NGO):
                src = table_hbm.at[idx_v.at[pl.ds(row0 + g * GCHUNK, GCHUNK)]]
                dst = rows_v.at[pl.ds(g * GCHUNK, GCHUNK), :]
                copies.append(pltpu.async_copy(src, dst, sem))
            for cp in copies:
                cp.wait()
            pltpu.sync_copy(rows_v, out_hbm.at[pl.ds(base + row0, OCHUNK), :])
            return carry

        lax.fori_loop(0, NOUTER, outer, 0)

    return k(idx_flat, table2d)


def kernel(sparse_inputs, tables):
    idx = sparse_inputs.astype(jnp.int32).reshape(ROWS)
    tab_t2 = jnp.transpose(tables, (0, 2, 1)).reshape(
        N_FIELDS * EMBED_DIM, VOCAB
    )
    tab_lin = _tc_detile(tab_t2).reshape(NQ * VOCAB * FQ, EMBED_DIM)
    out = _sc_gather(idx, tab_lin)
    return out.reshape(BATCH, N_FIELDS, EMBED_DIM)


# speedup vs baseline: 3.1826x; 1.1014x over previous
import functools

import jax
import jax.numpy as jnp
from jax import lax
from jax.experimental import pallas as pl
from jax.experimental.pallas import tpu as pltpu
from jax.experimental.pallas import tpu_sc as plsc

N_FIELDS = 26
VOCAB = 100000
EMBED_DIM = 32
BATCH = 16384

NC = 2
NS = 16
L = 16
NW = NC * NS

ROWS = BATCH * N_FIELDS
RPW = ROWS // NW
GCHUNK = 128
OCHUNK = 1024
NGO = OCHUNK // GCHUNK
NOUTER = RPW // OCHUNK

VB = 16384
NVB = (VOCAB + VB - 1) // VB
FQ = 4
NQ = (N_FIELDS + FQ - 1) // FQ
QW = FQ * EMBED_DIM


def _tc_detile(tab_t2):
    def body(i_ref, o_ref):
        o_ref[0] = i_ref[...].T

    return pl.pallas_call(
        body,
        grid=(NQ, NVB),
        in_specs=[
            pl.BlockSpec((QW, VB), lambda q, k: (q, k)),
        ],
        out_specs=pl.BlockSpec((1, VB, QW), lambda q, k: (q, k, 0)),
        out_shape=jax.ShapeDtypeStruct((NQ, VOCAB, QW), jnp.float32),
    )(tab_t2)


BPW = BATCH // NW
NBC = BPW // GCHUNK


def _sc_gather(idx_t, table2d):
    mesh = plsc.VectorSubcoreMesh(core_axis_name="c", subcore_axis_name="s")

    @functools.partial(
        pl.kernel,
        out_type=jax.ShapeDtypeStruct((N_FIELDS * BATCH, EMBED_DIM), jnp.float32),
        mesh=mesh,
        scratch_types=[
            pltpu.VMEM((N_FIELDS, BPW), jnp.int32),
            pltpu.VMEM((BPW, EMBED_DIM), jnp.float32),
            pltpu.SemaphoreType.DMA,
        ],
        compiler_params=pltpu.CompilerParams(use_tc_tiling_on_sc=False),
    )
    def k(idx_hbm, table_hbm, out_hbm, idx_v, rows_v, sem):
        wid = lax.axis_index("s") * NC + lax.axis_index("c")
        b0 = wid * BPW

        pltpu.sync_copy(idx_hbm.at[:, pl.ds(b0, BPW)], idx_v)

        def field(f, carry):
            off = lax.div(f, FQ) * (FQ * VOCAB) + lax.rem(f, FQ)

            def fix(i, c):
                sl = pl.ds(i * L, L)
                idx_v[f, sl] = idx_v[f, sl] * FQ + off
                return c

            lax.fori_loop(0, BPW // L, fix, 0)

            copies = []
            for g in range(NBC):
                src = table_hbm.at[idx_v.at[f, pl.ds(g * GCHUNK, GCHUNK)]]
                dst = rows_v.at[pl.ds(g * GCHUNK, GCHUNK), :]
                copies.append(pltpu.async_copy(src, dst, sem))
            for cp in copies:
                cp.wait()
            pltpu.sync_copy(
                rows_v, out_hbm.at[pl.ds(f * BATCH + b0, BPW), :]
            )
            return carry

        lax.fori_loop(0, N_FIELDS, field, 0)

    return k(idx_t, table2d)


def kernel(sparse_inputs, tables):
    idx_t = sparse_inputs.astype(jnp.int32).T
    tab_t2 = jnp.transpose(tables, (0, 2, 1)).reshape(
        N_FIELDS * EMBED_DIM, VOCAB
    )
    tab_lin = _tc_detile(tab_t2).reshape(NQ * VOCAB * FQ, EMBED_DIM)
    out = _sc_gather(idx_t, tab_lin)
    return jnp.transpose(
        out.reshape(N_FIELDS, BATCH, EMBED_DIM), (1, 0, 2)
    )


# speedup vs baseline: 3.2675x; 1.0267x over previous
import functools

import jax
import jax.numpy as jnp
from jax import lax
from jax.experimental import pallas as pl
from jax.experimental.pallas import tpu as pltpu
from jax.experimental.pallas import tpu_sc as plsc

N_FIELDS = 26
VOCAB = 100000
EMBED_DIM = 32
BATCH = 16384

NC = 2
NS = 16
L = 16
NW = NC * NS

ROWS = BATCH * N_FIELDS
RPW = ROWS // NW
GCHUNK = 128
OCHUNK = 1024
NGO = OCHUNK // GCHUNK
NOUTER = RPW // OCHUNK

VB = 16384
NVB = (VOCAB + VB - 1) // VB
FQ = 4
NQ = (N_FIELDS + FQ - 1) // FQ
QW = FQ * EMBED_DIM


def _tc_detile(tab_t2, q0, nq):
    def body(i_ref, o_ref):
        o_ref[0] = i_ref[...].T

    return pl.pallas_call(
        body,
        grid=(nq, NVB),
        in_specs=[
            pl.BlockSpec((QW, VB), lambda q, k: (q0 + q, k)),
        ],
        out_specs=pl.BlockSpec((1, VB, QW), lambda q, k: (q, k, 0)),
        out_shape=jax.ShapeDtypeStruct((nq, VOCAB, QW), jnp.float32),
    )(tab_t2)


BPW = BATCH // NW
NBC = BPW // GCHUNK


def _sc_gather(idx_t, table2d, nf):
    mesh = plsc.VectorSubcoreMesh(core_axis_name="c", subcore_axis_name="s")

    @functools.partial(
        pl.kernel,
        out_type=jax.ShapeDtypeStruct((nf * BATCH, EMBED_DIM), jnp.float32),
        mesh=mesh,
        scratch_types=[
            pltpu.VMEM((nf, BPW), jnp.int32),
            pltpu.VMEM((BPW, EMBED_DIM), jnp.float32),
            pltpu.SemaphoreType.DMA,
        ],
        compiler_params=pltpu.CompilerParams(use_tc_tiling_on_sc=False),
    )
    def k(idx_hbm, table_hbm, out_hbm, idx_v, rows_v, sem):
        wid = lax.axis_index("s") * NC + lax.axis_index("c")
        b0 = wid * BPW

        pltpu.sync_copy(idx_hbm.at[:, pl.ds(b0, BPW)], idx_v)

        def field(f, carry):
            off = lax.div(f, FQ) * (FQ * VOCAB) + lax.rem(f, FQ)

            def fix(i, c):
                sl = pl.ds(i * L, L)
                idx_v[f, sl] = idx_v[f, sl] * FQ + off
                return c

            lax.fori_loop(0, BPW // L, fix, 0)

            copies = []
            for g in range(NBC):
                src = table_hbm.at[idx_v.at[f, pl.ds(g * GCHUNK, GCHUNK)]]
                dst = rows_v.at[pl.ds(g * GCHUNK, GCHUNK), :]
                copies.append(pltpu.async_copy(src, dst, sem))
            for cp in copies:
                cp.wait()
            pltpu.sync_copy(
                rows_v, out_hbm.at[pl.ds(f * BATCH + b0, BPW), :]
            )
            return carry

        lax.fori_loop(0, nf, field, 0)

    return k(idx_t, table2d)


FSPLIT = 16


def kernel(sparse_inputs, tables):
    idx_t = sparse_inputs.astype(jnp.int32).T
    tab_t2 = jnp.transpose(tables, (0, 2, 1)).reshape(
        N_FIELDS * EMBED_DIM, VOCAB
    )
    tab_a = _tc_detile(tab_t2, 0, 4).reshape(4 * VOCAB * FQ, EMBED_DIM)
    out_a = _sc_gather(idx_t[:FSPLIT], tab_a, FSPLIT)
    tab_b = _tc_detile(tab_t2, 4, 3).reshape(3 * VOCAB * FQ, EMBED_DIM)
    out_b = _sc_gather(idx_t[FSPLIT:], tab_b, N_FIELDS - FSPLIT)
    out = jnp.concatenate(
        [
            out_a.reshape(FSPLIT, BATCH, EMBED_DIM),
            out_b.reshape(N_FIELDS - FSPLIT, BATCH, EMBED_DIM),
        ],
        axis=0,
    )
    return jnp.transpose(out, (1, 0, 2))
